# baseline (speedup 1.0000x reference)
import functools

import jax
import jax.numpy as jnp
from jax import lax
from jax.experimental import pallas as pl
from jax.experimental.pallas import tpu as pltpu
from jax.experimental.pallas import tpu_sc as plsc

N = 10000
D = 128
E = 320000
NW = 32
CHUNK = 128
C = 80
G = 8
NG = C // G
EPW = C * CHUNK
EPAD = NW * EPW
NSLICE = 632
NPAD = 16 * NSLICE

_MESH = dict(core_axis_name="c", subcore_axis_name="s")
_SLICES = [(0, 128), (128, 128), (256, 128), (384, 128), (512, 120)]


@functools.partial(
    pl.kernel,
    mesh=plsc.VectorSubcoreMesh(**_MESH),
    out_type=jax.ShapeDtypeStruct((2 * NPAD, D), jnp.float32),
    scratch_types=[
        pltpu.VMEM((2 * G, CHUNK), jnp.int32),
        pltpu.VMEM((2, CHUNK, D), jnp.float32),
        pltpu.VMEM_SHARED((NPAD, D), jnp.float32),
        pltpu.SemaphoreType.DMA,
    ],
)
def _seg_sum(table, eidx, zacc, acc_out, idx_v, rows_v, acc_sh, sem):
    c = lax.axis_index("c")
    s = lax.axis_index("s")
    wid = c * 16 + s
    r0 = s * NSLICE
    for t, sz in _SLICES:
        pltpu.sync_copy(zacc.at[pl.ds(r0 + t, sz)], rows_v.at[0, pl.ds(0, sz)])
        pltpu.sync_copy(rows_v.at[0, pl.ds(0, sz)], acc_sh.at[pl.ds(r0 + t, sz)])
    plsc.subcore_barrier()

    def body(g, carry):
        pltpu.sync_copy(eidx.at[pl.ds((wid * C + g * G) * 2, 2 * G)], idx_v)
        cps = {}
        cps[0] = pltpu.async_copy(table.at[idx_v.at[0]], rows_v.at[0], sem)
        for j in range(G):
            if j + 1 < G:
                cps[j + 1] = pltpu.async_copy(
                    table.at[idx_v.at[2 * (j + 1)]], rows_v.at[(j + 1) % 2], sem)
            cps[j].wait()
            pltpu.sync_copy(rows_v.at[j % 2], acc_sh.at[idx_v.at[2 * j + 1]],
                            add=True)
        return carry

    lax.fori_loop(0, NG, body, 0)

    plsc.subcore_barrier()
    o0 = c * NPAD + s * NSLICE
    for t, sz in _SLICES:
        pltpu.sync_copy(acc_sh.at[pl.ds(r0 + t, sz)], rows_v.at[0, pl.ds(0, sz)])
        pltpu.sync_copy(rows_v.at[0, pl.ds(0, sz)], acc_out.at[pl.ds(o0 + t, sz)])


@functools.partial(
    pl.kernel,
    mesh=plsc.VectorSubcoreMesh(**_MESH),
    out_type=jax.ShapeDtypeStruct((2 * NPAD, D), jnp.float32),
    scratch_types=[
        pltpu.VMEM((2 * G, CHUNK), jnp.int32),
        pltpu.VMEM((CHUNK, D), jnp.float32),
        pltpu.VMEM_SHARED((NPAD, D), jnp.float32),
        pltpu.SemaphoreType.DMA,
    ],
)
def _deg_sum(eidx, zacc, ones, deg_out, idx_v, ones_v, deg_sh, sem):
    c = lax.axis_index("c")
    s = lax.axis_index("s")
    wid = c * 16 + s
    r0 = s * NSLICE
    for t, sz in _SLICES:
        pltpu.sync_copy(zacc.at[pl.ds(r0 + t, sz)], ones_v.at[pl.ds(0, sz)])
        pltpu.sync_copy(ones_v.at[pl.ds(0, sz)], deg_sh.at[pl.ds(r0 + t, sz)])
    pltpu.sync_copy(ones, ones_v)
    plsc.subcore_barrier()

    def body(g, carry):
        pltpu.sync_copy(eidx.at[pl.ds((wid * C + g * G) * 2, 2 * G)], idx_v)
        cps = [pltpu.async_copy(ones_v, deg_sh.at[idx_v.at[2 * j + 1]], sem,
                                add=True)
               for j in range(G)]
        for cp in cps:
            cp.wait()
        return carry

    lax.fori_loop(0, NG, body, 0)

    plsc.subcore_barrier()
    o0 = c * NPAD + s * NSLICE
    for t, sz in _SLICES:
        pltpu.sync_copy(deg_sh.at[pl.ds(r0 + t, sz)], ones_v.at[pl.ds(0, sz)])
        pltpu.sync_copy(ones_v.at[pl.ds(0, sz)], deg_out.at[pl.ds(o0 + t, sz)])


_MMB = 2000


def _mm2_body(x_ref, wl_ref, wr_ref, p_ref, r_ref):
    x = x_ref[...]
    p_ref[...] = jnp.dot(x, wl_ref[...], preferred_element_type=jnp.float32)
    r_ref[...] = jnp.dot(x, wr_ref[...], preferred_element_type=jnp.float32)


def _mm2(x, wl, wr):
    return pl.pallas_call(
        _mm2_body,
        grid=(N // _MMB,),
        in_specs=[
            pl.BlockSpec((_MMB, D), lambda i: (i, 0)),
            pl.BlockSpec((D, D), lambda i: (0, 0)),
            pl.BlockSpec((D, D), lambda i: (0, 0)),
        ],
        out_specs=[pl.BlockSpec((_MMB, D), lambda i: (i, 0))] * 2,
        out_shape=[jax.ShapeDtypeStruct((N, D), jnp.float32)] * 2,
    )(x, wl, wr)


def _fuse_body(acc_ref, deg_ref, r1_ref, b_ref, wl_ref, wr_ref, p2_ref, r2_ref):
    a = acc_ref[0] + acc_ref[1]
    dcol = deg_ref[0, :, :1] + deg_ref[1, :, :1]
    inv = 1.0 / jnp.maximum(dcol, 1.0)
    h = jnp.maximum(a * inv + b_ref[...] + r1_ref[...], 0.0)
    p2_ref[...] = jnp.dot(h, wl_ref[...], preferred_element_type=jnp.float32)
    r2_ref[...] = jnp.dot(h, wr_ref[...], preferred_element_type=jnp.float32)


def _fuse(acc, deg, r1, b1, wl, wr):
    return pl.pallas_call(
        _fuse_body,
        grid=(N // _MMB,),
        in_specs=[
            pl.BlockSpec((2, _MMB, D), lambda i: (0, i, 0)),
            pl.BlockSpec((2, _MMB, D), lambda i: (0, i, 0)),
            pl.BlockSpec((_MMB, D), lambda i: (i, 0)),
            pl.BlockSpec((1, D), lambda i: (0, 0)),
            pl.BlockSpec((D, D), lambda i: (0, 0)),
            pl.BlockSpec((D, D), lambda i: (0, 0)),
        ],
        out_specs=[pl.BlockSpec((_MMB, D), lambda i: (i, 0))] * 2,
        out_shape=[jax.ShapeDtypeStruct((N, D), jnp.float32)] * 2,
    )(acc, deg, r1, b1, wl, wr)


def _final_body(acc_ref, deg_ref, r2_ref, b_ref, out_ref):
    a = acc_ref[0] + acc_ref[1]
    dcol = deg_ref[0, :, :1] + deg_ref[1, :, :1]
    inv = 1.0 / jnp.maximum(dcol, 1.0)
    out_ref[...] = a * inv + b_ref[...] + r2_ref[...]


def _final(acc, deg, r2, b2):
    return pl.pallas_call(
        _final_body,
        grid=(N // _MMB,),
        in_specs=[
            pl.BlockSpec((2, _MMB, D), lambda i: (0, i, 0)),
            pl.BlockSpec((2, _MMB, D), lambda i: (0, i, 0)),
            pl.BlockSpec((_MMB, D), lambda i: (i, 0)),
            pl.BlockSpec((1, D), lambda i: (0, 0)),
        ],
        out_specs=pl.BlockSpec((_MMB, D), lambda i: (i, 0)),
        out_shape=jax.ShapeDtypeStruct((N, D), jnp.float32),
    )(acc, deg, r2, b2)


def kernel(x, edge_index, W1_l, b1_l, W1_r, W2_l, b2_l, W2_r):
    pad = EPAD - E
    src = jnp.concatenate(
        [edge_index[0].astype(jnp.int32), jnp.zeros((pad,), jnp.int32)])
    dst = jnp.concatenate(
        [edge_index[1].astype(jnp.int32), jnp.full((pad,), N, jnp.int32)])
    srcr = src.reshape(NW * C, CHUNK)
    wof = (jnp.arange(NW * C, dtype=jnp.int32)[:, None] // C >= 16)
    srcr = srcr + wof.astype(jnp.int32) * N
    eidx = jnp.stack([srcr, dst.reshape(NW * C, CHUNK)],
                     axis=1).reshape(2 * NW * C, CHUNK)
    zacc = jnp.zeros((NPAD, D), jnp.float32)
    ones = jnp.ones((CHUNK, D), jnp.float32)
    b1 = b1_l.reshape(1, D)
    b2 = b2_l.reshape(1, D)

    degp = _deg_sum(eidx, zacc, ones).reshape(2, NPAD, D)
    p1, r1 = _mm2(x, W1_l, W1_r)
    p1d = jnp.concatenate([p1, p1], axis=0)
    acc1 = _seg_sum(p1d, eidx, zacc).reshape(2, NPAD, D)
    p2, r2 = _fuse(acc1, degp, r1, b1, W2_l, W2_r)
    p2d = jnp.concatenate([p2, p2], axis=0)
    acc2 = _seg_sum(p2d, eidx, zacc).reshape(2, NPAD, D)
    return _final(acc2, degp, r2, b2)


# speedup vs baseline: 2.5980x; 2.5980x over previous
import functools

import jax
import jax.numpy as jnp
from jax import lax
from jax.experimental import pallas as pl
from jax.experimental.pallas import tpu as pltpu
from jax.experimental.pallas import tpu_sc as plsc

N = 10000
D = 128
E = 320000
NW = 32
CHUNK = 128
C = 80
G = 8
NG = C // G
EPW = C * CHUNK
EPAD = NW * EPW
NSLICE = 632
NPAD = 16 * NSLICE

_MESH = dict(core_axis_name="c", subcore_axis_name="s")
_SLICES = [(0, 128), (128, 128), (256, 128), (384, 128), (512, 120)]


@functools.partial(
    pl.kernel,
    mesh=plsc.VectorSubcoreMesh(**_MESH),
    out_type=jax.ShapeDtypeStruct((2 * NPAD, D), jnp.float32),
    scratch_types=[
        pltpu.VMEM((2 * G, CHUNK), jnp.int32),
        pltpu.VMEM((2, CHUNK, D), jnp.float32),
        pltpu.VMEM_SHARED((NPAD, D), jnp.float32),
        pltpu.SemaphoreType.DMA,
    ],
)
def _seg_sum(table, eidx, zacc, acc_out, idx_v, rows_v, acc_sh, sem):
    c = lax.axis_index("c")
    s = lax.axis_index("s")
    wid = c * 16 + s
    r0 = s * NSLICE
    for t, sz in _SLICES:
        pltpu.sync_copy(zacc.at[pl.ds(r0 + t, sz)], rows_v.at[0, pl.ds(0, sz)])
        pltpu.sync_copy(rows_v.at[0, pl.ds(0, sz)], acc_sh.at[pl.ds(r0 + t, sz)])
    plsc.subcore_barrier()

    def body(g, carry):
        pltpu.sync_copy(eidx.at[pl.ds((wid * C + g * G) * 2, 2 * G)], idx_v)
        cps = {}
        cps[0] = pltpu.async_copy(table.at[idx_v.at[0]], rows_v.at[0], sem)
        for j in range(G):
            if j + 1 < G:
                cps[j + 1] = pltpu.async_copy(
                    table.at[idx_v.at[2 * (j + 1)]], rows_v.at[(j + 1) % 2], sem)
            cps[j].wait()
            pltpu.sync_copy(rows_v.at[j % 2], acc_sh.at[idx_v.at[2 * j + 1]],
                            add=True)
        return carry

    lax.fori_loop(0, NG, body, 0)

    plsc.subcore_barrier()
    o0 = c * NPAD + s * NSLICE
    for t, sz in _SLICES:
        pltpu.sync_copy(acc_sh.at[pl.ds(r0 + t, sz)], rows_v.at[0, pl.ds(0, sz)])
        pltpu.sync_copy(rows_v.at[0, pl.ds(0, sz)], acc_out.at[pl.ds(o0 + t, sz)])


@functools.partial(
    pl.kernel,
    mesh=plsc.VectorSubcoreMesh(**_MESH),
    out_type=jax.ShapeDtypeStruct((2 * NPAD, D), jnp.float32),
    scratch_types=[
        pltpu.VMEM((2 * G, CHUNK), jnp.int32),
        pltpu.VMEM((CHUNK, D), jnp.float32),
        pltpu.VMEM_SHARED((NPAD, D), jnp.float32),
        pltpu.SemaphoreType.DMA,
    ],
)
def _deg_sum(eidx, zacc, ones, deg_out, idx_v, ones_v, deg_sh, sem):
    c = lax.axis_index("c")
    s = lax.axis_index("s")
    wid = c * 16 + s
    r0 = s * NSLICE
    for t, sz in _SLICES:
        pltpu.sync_copy(zacc.at[pl.ds(r0 + t, sz)], ones_v.at[pl.ds(0, sz)])
        pltpu.sync_copy(ones_v.at[pl.ds(0, sz)], deg_sh.at[pl.ds(r0 + t, sz)])
    pltpu.sync_copy(ones, ones_v)
    plsc.subcore_barrier()

    def body(g, carry):
        pltpu.sync_copy(eidx.at[pl.ds((wid * C + g * G) * 2, 2 * G)], idx_v)
        cps = [pltpu.async_copy(ones_v, deg_sh.at[idx_v.at[2 * j + 1]], sem,
                                add=True)
               for j in range(G)]
        for cp in cps:
            cp.wait()
        return carry

    lax.fori_loop(0, NG, body, 0)

    plsc.subcore_barrier()
    o0 = c * NPAD + s * NSLICE
    for t, sz in _SLICES:
        pltpu.sync_copy(deg_sh.at[pl.ds(r0 + t, sz)], ones_v.at[pl.ds(0, sz)])
        pltpu.sync_copy(ones_v.at[pl.ds(0, sz)], deg_out.at[pl.ds(o0 + t, sz)])


_MMB = 2000


def _mm2_body(x_ref, wl_ref, wr_ref, p_ref, r_ref):
    x = x_ref[...]
    p_ref[...] = jnp.dot(x, wl_ref[...], preferred_element_type=jnp.float32)
    r_ref[...] = jnp.dot(x, wr_ref[...], preferred_element_type=jnp.float32)


def _mm2(x, wl, wr):
    return pl.pallas_call(
        _mm2_body,
        grid=(N // _MMB,),
        in_specs=[
            pl.BlockSpec((_MMB, D), lambda i: (i, 0)),
            pl.BlockSpec((D, D), lambda i: (0, 0)),
            pl.BlockSpec((D, D), lambda i: (0, 0)),
        ],
        out_specs=[pl.BlockSpec((_MMB, D), lambda i: (i, 0))] * 2,
        out_shape=[jax.ShapeDtypeStruct((N, D), jnp.float32)] * 2,
    )(x, wl, wr)


def _fuse_body(acc_ref, deg_ref, r1_ref, b_ref, wl_ref, wr_ref, p2_ref, r2_ref):
    a = acc_ref[0] + acc_ref[1]
    dcol = deg_ref[0, :, :1] + deg_ref[1, :, :1]
    inv = 1.0 / jnp.maximum(dcol, 1.0)
    h = jnp.maximum(a * inv + b_ref[...] + r1_ref[...], 0.0)
    p2_ref[...] = jnp.dot(h, wl_ref[...], preferred_element_type=jnp.float32)
    r2_ref[...] = jnp.dot(h, wr_ref[...], preferred_element_type=jnp.float32)


def _fuse(acc, deg, r1, b1, wl, wr):
    return pl.pallas_call(
        _fuse_body,
        grid=(N // _MMB,),
        in_specs=[
            pl.BlockSpec((2, _MMB, D), lambda i: (0, i, 0)),
            pl.BlockSpec((2, _MMB, D), lambda i: (0, i, 0)),
            pl.BlockSpec((_MMB, D), lambda i: (i, 0)),
            pl.BlockSpec((1, D), lambda i: (0, 0)),
            pl.BlockSpec((D, D), lambda i: (0, 0)),
            pl.BlockSpec((D, D), lambda i: (0, 0)),
        ],
        out_specs=[pl.BlockSpec((_MMB, D), lambda i: (i, 0))] * 2,
        out_shape=[jax.ShapeDtypeStruct((N, D), jnp.float32)] * 2,
    )(acc, deg, r1, b1, wl, wr)


def _final_body(acc_ref, deg_ref, r2_ref, b_ref, out_ref):
    a = acc_ref[0] + acc_ref[1]
    dcol = deg_ref[0, :, :1] + deg_ref[1, :, :1]
    inv = 1.0 / jnp.maximum(dcol, 1.0)
    out_ref[...] = a * inv + b_ref[...] + r2_ref[...]


def _final(acc, deg, r2, b2):
    return pl.pallas_call(
        _final_body,
        grid=(N // _MMB,),
        in_specs=[
            pl.BlockSpec((2, _MMB, D), lambda i: (0, i, 0)),
            pl.BlockSpec((2, _MMB, D), lambda i: (0, i, 0)),
            pl.BlockSpec((_MMB, D), lambda i: (i, 0)),
            pl.BlockSpec((1, D), lambda i: (0, 0)),
        ],
        out_specs=pl.BlockSpec((_MMB, D), lambda i: (i, 0)),
        out_shape=jax.ShapeDtypeStruct((N, D), jnp.float32),
    )(acc, deg, r2, b2)


def kernel(x, edge_index, W1_l, b1_l, W1_r, W2_l, b2_l, W2_r):
    pad = EPAD - E
    ar = jnp.arange(pad, dtype=jnp.int32)
    src = jnp.concatenate([edge_index[0].astype(jnp.int32), ar % N])
    dst = jnp.concatenate(
        [edge_index[1].astype(jnp.int32), N + ar % (NPAD - N)])
    srcr = src.reshape(NW * C, CHUNK)
    wof = (jnp.arange(NW * C, dtype=jnp.int32)[:, None] // C >= 16)
    srcr = srcr + wof.astype(jnp.int32) * N
    eidx = jnp.stack([srcr, dst.reshape(NW * C, CHUNK)],
                     axis=1).reshape(2 * NW * C, CHUNK)
    zacc = jnp.zeros((NPAD, D), jnp.float32)
    ones = jnp.ones((CHUNK, D), jnp.float32)
    b1 = b1_l.reshape(1, D)
    b2 = b2_l.reshape(1, D)

    degp = _deg_sum(eidx, zacc, ones).reshape(2, NPAD, D)
    p1, r1 = _mm2(x, W1_l, W1_r)
    p1d = jnp.concatenate([p1, p1], axis=0)
    acc1 = _seg_sum(p1d, eidx, zacc).reshape(2, NPAD, D)
    p2, r2 = _fuse(acc1, degp, r1, b1, W2_l, W2_r)
    p2d = jnp.concatenate([p2, p2], axis=0)
    acc2 = _seg_sum(p2d, eidx, zacc).reshape(2, NPAD, D)
    return _final(acc2, degp, r2, b2)


# speedup vs baseline: 2.7756x; 1.0683x over previous
import functools

import jax
import jax.numpy as jnp
from jax import lax
from jax.experimental import pallas as pl
from jax.experimental.pallas import tpu as pltpu
from jax.experimental.pallas import tpu_sc as plsc

N = 10000
D = 128
E = 320000
NW = 32
CHUNK = 128
C = 80
G = 8
NG = C // G
EPW = C * CHUNK
EPAD = NW * EPW
NSLICE = 632
NPAD = 16 * NSLICE

_MESH = dict(core_axis_name="c", subcore_axis_name="s")
_SLICES = [(0, 128), (128, 128), (256, 128), (384, 128), (512, 120)]


@functools.partial(
    pl.kernel,
    mesh=plsc.VectorSubcoreMesh(**_MESH),
    out_type=jax.ShapeDtypeStruct((2 * NPAD, D), jnp.float32),
    scratch_types=[
        pltpu.VMEM((2 * G, CHUNK), jnp.int32),
        pltpu.VMEM((2, CHUNK, D), jnp.float32),
        pltpu.VMEM_SHARED((NPAD, D), jnp.float32),
        pltpu.SemaphoreType.DMA,
    ],
)
def _seg_sum(table, eidx, zacc, acc_out, idx_v, rows_v, acc_sh, sem):
    c = lax.axis_index("c")
    s = lax.axis_index("s")
    wid = c * 16 + s
    r0 = s * NSLICE
    for t, sz in _SLICES:
        pltpu.sync_copy(zacc.at[pl.ds(r0 + t, sz)], rows_v.at[0, pl.ds(0, sz)])
        pltpu.sync_copy(rows_v.at[0, pl.ds(0, sz)], acc_sh.at[pl.ds(r0 + t, sz)])
    plsc.subcore_barrier()

    def body(g, carry):
        pltpu.sync_copy(eidx.at[pl.ds((wid * C + g * G) * 2, 2 * G)], idx_v)
        cps = {}
        cps[0] = pltpu.async_copy(table.at[idx_v.at[0]], rows_v.at[0], sem)
        for j in range(G):
            if j + 1 < G:
                cps[j + 1] = pltpu.async_copy(
                    table.at[idx_v.at[2 * (j + 1)]], rows_v.at[(j + 1) % 2], sem)
            cps[j].wait()
            pltpu.sync_copy(rows_v.at[j % 2], acc_sh.at[idx_v.at[2 * j + 1]],
                            add=True)
        return carry

    lax.fori_loop(0, NG, body, 0)

    plsc.subcore_barrier()
    o0 = c * NPAD + s * NSLICE
    for t, sz in _SLICES:
        pltpu.sync_copy(acc_sh.at[pl.ds(r0 + t, sz)], rows_v.at[0, pl.ds(0, sz)])
        pltpu.sync_copy(rows_v.at[0, pl.ds(0, sz)], acc_out.at[pl.ds(o0 + t, sz)])


@functools.partial(
    pl.kernel,
    mesh=plsc.VectorSubcoreMesh(**_MESH),
    out_type=jax.ShapeDtypeStruct((2 * NPAD, D), jnp.float32),
    scratch_types=[
        pltpu.VMEM((2 * G, CHUNK), jnp.int32),
        pltpu.VMEM((CHUNK, D), jnp.float32),
        pltpu.VMEM_SHARED((NPAD, D), jnp.float32),
        pltpu.SemaphoreType.DMA,
    ],
)
def _deg_sum(eidx, zacc, ones, deg_out, idx_v, ones_v, deg_sh, sem):
    c = lax.axis_index("c")
    s = lax.axis_index("s")
    wid = c * 16 + s
    r0 = s * NSLICE
    for t, sz in _SLICES:
        pltpu.sync_copy(zacc.at[pl.ds(r0 + t, sz)], ones_v.at[pl.ds(0, sz)])
        pltpu.sync_copy(ones_v.at[pl.ds(0, sz)], deg_sh.at[pl.ds(r0 + t, sz)])
    pltpu.sync_copy(ones, ones_v)
    plsc.subcore_barrier()

    def body(g, carry):
        pltpu.sync_copy(eidx.at[pl.ds((wid * C + g * G) * 2, 2 * G)], idx_v)
        cps = [pltpu.async_copy(ones_v, deg_sh.at[idx_v.at[2 * j + 1]], sem,
                                add=True)
               for j in range(G)]
        for cp in cps:
            cp.wait()
        return carry

    lax.fori_loop(0, NG, body, 0)

    plsc.subcore_barrier()
    o0 = c * NPAD + s * NSLICE
    for t, sz in _SLICES:
        pltpu.sync_copy(deg_sh.at[pl.ds(r0 + t, sz)], ones_v.at[pl.ds(0, sz)])
        pltpu.sync_copy(ones_v.at[pl.ds(0, sz)], deg_out.at[pl.ds(o0 + t, sz)])


_MMB = 2000


def _mm2_body(x_ref, wl_ref, wr_ref, p_ref, r_ref):
    x = x_ref[...]
    p_ref[...] = jnp.dot(x, wl_ref[...], preferred_element_type=jnp.float32)
    r_ref[...] = jnp.dot(x, wr_ref[...], preferred_element_type=jnp.float32)


def _mm2(x, wl, wr):
    return pl.pallas_call(
        _mm2_body,
        grid=(N // _MMB,),
        in_specs=[
            pl.BlockSpec((_MMB, D), lambda i: (i, 0)),
            pl.BlockSpec((D, D), lambda i: (0, 0)),
            pl.BlockSpec((D, D), lambda i: (0, 0)),
        ],
        out_specs=[pl.BlockSpec((_MMB, D), lambda i: (i, 0))] * 2,
        out_shape=[jax.ShapeDtypeStruct((N, D), jnp.float32)] * 2,
    )(x, wl, wr)


def _fuse_body(acc_ref, deg_ref, r1_ref, b_ref, wl_ref, wr_ref, p2_ref, r2_ref):
    a = acc_ref[0] + acc_ref[1]
    dcol = deg_ref[0, :, :1] + deg_ref[1, :, :1]
    inv = 1.0 / jnp.maximum(dcol, 1.0)
    h = jnp.maximum(a * inv + b_ref[...] + r1_ref[...], 0.0)
    p2_ref[...] = jnp.dot(h, wl_ref[...], preferred_element_type=jnp.float32)
    r2_ref[...] = jnp.dot(h, wr_ref[...], preferred_element_type=jnp.float32)


def _fuse(acc, deg, r1, b1, wl, wr):
    return pl.pallas_call(
        _fuse_body,
        grid=(N // _MMB,),
        in_specs=[
            pl.BlockSpec((2, _MMB, D), lambda i: (0, i, 0)),
            pl.BlockSpec((2, _MMB, D), lambda i: (0, i, 0)),
            pl.BlockSpec((_MMB, D), lambda i: (i, 0)),
            pl.BlockSpec((1, D), lambda i: (0, 0)),
            pl.BlockSpec((D, D), lambda i: (0, 0)),
            pl.BlockSpec((D, D), lambda i: (0, 0)),
        ],
        out_specs=[pl.BlockSpec((_MMB, D), lambda i: (i, 0))] * 2,
        out_shape=[jax.ShapeDtypeStruct((N, D), jnp.float32)] * 2,
    )(acc, deg, r1, b1, wl, wr)


def _final_body(acc_ref, deg_ref, r2_ref, b_ref, out_ref):
    a = acc_ref[0] + acc_ref[1]
    dcol = deg_ref[0, :, :1] + deg_ref[1, :, :1]
    inv = 1.0 / jnp.maximum(dcol, 1.0)
    out_ref[...] = a * inv + b_ref[...] + r2_ref[...]


def _final(acc, deg, r2, b2):
    return pl.pallas_call(
        _final_body,
        grid=(N // _MMB,),
        in_specs=[
            pl.BlockSpec((2, _MMB, D), lambda i: (0, i, 0)),
            pl.BlockSpec((2, _MMB, D), lambda i: (0, i, 0)),
            pl.BlockSpec((_MMB, D), lambda i: (i, 0)),
            pl.BlockSpec((1, D), lambda i: (0, 0)),
        ],
        out_specs=pl.BlockSpec((_MMB, D), lambda i: (i, 0)),
        out_shape=jax.ShapeDtypeStruct((N, D), jnp.float32),
    )(acc, deg, r2, b2)


def kernel(x, edge_index, W1_l, b1_l, W1_r, W2_l, b2_l, W2_r):
    pad = EPAD - E
    ar = jnp.arange(pad, dtype=jnp.int32)
    src = jnp.concatenate([edge_index[0].astype(jnp.int32), ar % N])
    dst = jnp.concatenate(
        [edge_index[1].astype(jnp.int32), N + ar % (NPAD - N)])
    eidx = jnp.stack([src.reshape(NW * C, CHUNK), dst.reshape(NW * C, CHUNK)],
                     axis=1).reshape(2 * NW * C, CHUNK)
    zacc = jnp.zeros((NPAD, D), jnp.float32)
    ones = jnp.ones((CHUNK, D), jnp.float32)
    b1 = b1_l.reshape(1, D)
    b2 = b2_l.reshape(1, D)

    degp = _deg_sum(eidx, zacc, ones).reshape(2, NPAD, D)
    p1, r1 = _mm2(x, W1_l, W1_r)
    acc1 = _seg_sum(p1, eidx, zacc).reshape(2, NPAD, D)
    p2, r2 = _fuse(acc1, degp, r1, b1, W2_l, W2_r)
    acc2 = _seg_sum(p2, eidx, zacc).reshape(2, NPAD, D)
    return _final(acc2, degp, r2, b2)


# speedup vs baseline: 2.8232x; 1.0171x over previous
import functools

import jax
import jax.numpy as jnp
from jax import lax
from jax.experimental import pallas as pl
from jax.experimental.pallas import tpu as pltpu
from jax.experimental.pallas import tpu_sc as plsc

N = 10000
D = 128
E = 320000
NW = 32
CHUNK = 128
C = 80
G = 8
NG = C // G
EPW = C * CHUNK
EPAD = NW * EPW
NSLICE = 632
NPAD = 16 * NSLICE

_MESH = dict(core_axis_name="c", subcore_axis_name="s")
_SLICES = [(0, 128), (128, 128), (256, 128), (384, 128), (512, 120)]


@functools.partial(
    pl.kernel,
    mesh=plsc.VectorSubcoreMesh(**_MESH),
    out_type=jax.ShapeDtypeStruct((2 * NPAD, D), jnp.float32),
    scratch_types=[
        pltpu.VMEM((2 * G, CHUNK), jnp.int32),
        pltpu.VMEM((2, CHUNK, D), jnp.float32),
        pltpu.VMEM_SHARED((NPAD, D), jnp.float32),
        pltpu.SemaphoreType.DMA,
    ],
)
def _seg_sum(table, eidx, zacc, acc_out, idx_v, rows_v, acc_sh, sem):
    c = lax.axis_index("c")
    s = lax.axis_index("s")
    wid = c * 16 + s
    r0 = s * NSLICE
    for t, sz in _SLICES:
        pltpu.sync_copy(zacc.at[pl.ds(r0 + t, sz)], rows_v.at[0, pl.ds(0, sz)])
        pltpu.sync_copy(rows_v.at[0, pl.ds(0, sz)], acc_sh.at[pl.ds(r0 + t, sz)])
    plsc.subcore_barrier()

    def body(g, carry):
        pltpu.sync_copy(eidx.at[pl.ds((wid * C + g * G) * 2, 2 * G)], idx_v)
        cps = {}
        cps[0] = pltpu.async_copy(table.at[idx_v.at[0]], rows_v.at[0], sem)
        for j in range(G):
            if j + 1 < G:
                cps[j + 1] = pltpu.async_copy(
                    table.at[idx_v.at[2 * (j + 1)]], rows_v.at[(j + 1) % 2], sem)
            cps[j].wait()
            pltpu.sync_copy(rows_v.at[j % 2], acc_sh.at[idx_v.at[2 * j + 1]],
                            add=True)
        return carry

    lax.fori_loop(0, NG, body, 0)

    plsc.subcore_barrier()
    o0 = c * NPAD + s * NSLICE
    for t, sz in _SLICES:
        pltpu.sync_copy(acc_sh.at[pl.ds(r0 + t, sz)], rows_v.at[0, pl.ds(0, sz)])
        pltpu.sync_copy(rows_v.at[0, pl.ds(0, sz)], acc_out.at[pl.ds(o0 + t, sz)])


@functools.partial(
    pl.kernel,
    mesh=plsc.VectorSubcoreMesh(**_MESH),
    out_type=jax.ShapeDtypeStruct((2 * NPAD, D), jnp.float32),
    scratch_types=[
        pltpu.VMEM((2 * G, CHUNK), jnp.int32),
        pltpu.VMEM((CHUNK, D), jnp.float32),
        pltpu.VMEM_SHARED((NPAD, D), jnp.float32),
        pltpu.SemaphoreType.DMA,
    ],
)
def _deg_sum(eidx, zacc, ones, deg_out, idx_v, ones_v, deg_sh, sem):
    c = lax.axis_index("c")
    s = lax.axis_index("s")
    wid = c * 16 + s
    r0 = s * NSLICE
    for t, sz in _SLICES:
        pltpu.sync_copy(zacc.at[pl.ds(r0 + t, sz)], ones_v.at[pl.ds(0, sz)])
        pltpu.sync_copy(ones_v.at[pl.ds(0, sz)], deg_sh.at[pl.ds(r0 + t, sz)])
    pltpu.sync_copy(ones, ones_v)
    plsc.subcore_barrier()

    def body(g, carry):
        pltpu.sync_copy(eidx.at[pl.ds((wid * C + g * G) * 2, 2 * G)], idx_v)
        cps = [pltpu.async_copy(ones_v, deg_sh.at[idx_v.at[2 * j + 1]], sem,
                                add=True)
               for j in range(G)]
        for cp in cps:
            cp.wait()
        return carry

    lax.fori_loop(0, NG, body, 0)

    plsc.subcore_barrier()
    o0 = c * NPAD + s * NSLICE
    for t, sz in _SLICES:
        pltpu.sync_copy(deg_sh.at[pl.ds(r0 + t, sz)], ones_v.at[pl.ds(0, sz)])
        pltpu.sync_copy(ones_v.at[pl.ds(0, sz)], deg_out.at[pl.ds(o0 + t, sz)])


_MMB = 2000


def _mm2_body(x_ref, wl_ref, wr_ref, p_ref, r_ref):
    x = x_ref[...]
    p_ref[...] = jnp.dot(x, wl_ref[...], preferred_element_type=jnp.float32)
    r_ref[...] = jnp.dot(x, wr_ref[...], preferred_element_type=jnp.float32)


def _mm2(x, wl, wr):
    return pl.pallas_call(
        _mm2_body,
        grid=(N // _MMB,),
        in_specs=[
            pl.BlockSpec((_MMB, D), lambda i: (i, 0)),
            pl.BlockSpec((D, D), lambda i: (0, 0)),
            pl.BlockSpec((D, D), lambda i: (0, 0)),
        ],
        out_specs=[pl.BlockSpec((_MMB, D), lambda i: (i, 0))] * 2,
        out_shape=[jax.ShapeDtypeStruct((N, D), jnp.float32)] * 2,
    )(x, wl, wr)


def _fuse_body(acc_ref, deg_ref, r1_ref, b_ref, wl_ref, wr_ref, p2_ref, r2_ref):
    a = acc_ref[0] + acc_ref[1]
    dcol = deg_ref[0, :, :1] + deg_ref[1, :, :1]
    inv = 1.0 / jnp.maximum(dcol, 1.0)
    h = jnp.maximum(a * inv + b_ref[...] + r1_ref[...], 0.0)
    p2_ref[...] = jnp.dot(h, wl_ref[...], preferred_element_type=jnp.float32)
    r2_ref[...] = jnp.dot(h, wr_ref[...], preferred_element_type=jnp.float32)


def _fuse(acc, deg, r1, b1, wl, wr):
    return pl.pallas_call(
        _fuse_body,
        grid=(N // _MMB,),
        in_specs=[
            pl.BlockSpec((2, _MMB, D), lambda i: (0, i, 0)),
            pl.BlockSpec((2, _MMB, D), lambda i: (0, i, 0)),
            pl.BlockSpec((_MMB, D), lambda i: (i, 0)),
            pl.BlockSpec((1, D), lambda i: (0, 0)),
            pl.BlockSpec((D, D), lambda i: (0, 0)),
            pl.BlockSpec((D, D), lambda i: (0, 0)),
        ],
        out_specs=[pl.BlockSpec((_MMB, D), lambda i: (i, 0))] * 2,
        out_shape=[jax.ShapeDtypeStruct((N, D), jnp.float32)] * 2,
    )(acc, deg, r1, b1, wl, wr)


def _final_body(acc_ref, deg_ref, r2_ref, b_ref, out_ref):
    a = acc_ref[0] + acc_ref[1]
    dcol = deg_ref[0, :, :1] + deg_ref[1, :, :1]
    inv = 1.0 / jnp.maximum(dcol, 1.0)
    out_ref[...] = a * inv + b_ref[...] + r2_ref[...]


def _final(acc, deg, r2, b2):
    return pl.pallas_call(
        _final_body,
        grid=(N // _MMB,),
        in_specs=[
            pl.BlockSpec((2, _MMB, D), lambda i: (0, i, 0)),
            pl.BlockSpec((2, _MMB, D), lambda i: (0, i, 0)),
            pl.BlockSpec((_MMB, D), lambda i: (i, 0)),
            pl.BlockSpec((1, D), lambda i: (0, 0)),
        ],
        out_specs=pl.BlockSpec((_MMB, D), lambda i: (i, 0)),
        out_shape=jax.ShapeDtypeStruct((N, D), jnp.float32),
    )(acc, deg, r2, b2)


def kernel(x, edge_index, W1_l, b1_l, W1_r, W2_l, b2_l, W2_r):
    pad = EPAD - E
    ar = jnp.arange(pad, dtype=jnp.int32)
    src = jnp.concatenate([edge_index[0].astype(jnp.int32), ar % N])
    dst = jnp.concatenate(
        [edge_index[1].astype(jnp.int32), N + ar % (NPAD - N)])
    eidx = jnp.stack([src.reshape(NW * C, CHUNK), dst.reshape(NW * C, CHUNK)],
                     axis=1).reshape(2 * NW * C, CHUNK)
    zacc = jnp.zeros((NPAD, D), jnp.float32)
    ones = jnp.ones((CHUNK, D), jnp.float32)
    b1 = b1_l.reshape(1, D)
    b2 = b2_l.reshape(1, D)

    degr = _deg_sum(eidx, zacc, ones)
    degp = degr.reshape(2, NPAD, D)
    p1, r1 = _mm2(x, W1_l, W1_r)
    marker = (degr[0, 0] * 0.0).astype(jnp.int32)
    acc1 = _seg_sum(p1, eidx + marker, zacc).reshape(2, NPAD, D)
    p2, r2 = _fuse(acc1, degp, r1, b1, W2_l, W2_r)
    acc2 = _seg_sum(p2, eidx, zacc).reshape(2, NPAD, D)
    return _final(acc2, degp, r2, b2)


# speedup vs baseline: 2.9555x; 1.0469x over previous
import functools

import jax
import jax.numpy as jnp
from jax import lax
from jax.experimental import pallas as pl
from jax.experimental.pallas import tpu as pltpu
from jax.experimental.pallas import tpu_sc as plsc

N = 10000
D = 128
E = 320000
NW = 32
CHUNK = 128
C = 80
G = 8
NG = C // G
EPW = C * CHUNK
EPAD = NW * EPW
NSLICE = 632
NPAD = 16 * NSLICE

_MESH = dict(core_axis_name="c", subcore_axis_name="s")
_SLICES = [(0, 128), (128, 128), (256, 128), (384, 128), (512, 120)]
_ZSLICES = [(64 * i, 64) for i in range(9)] + [(576, 56)]


@functools.partial(
    pl.kernel,
    mesh=plsc.VectorSubcoreMesh(**_MESH),
    out_type=jax.ShapeDtypeStruct((2 * NPAD, D), jnp.float32),
    scratch_types=[
        pltpu.VMEM((2 * G, CHUNK), jnp.int32),
        pltpu.VMEM((2, CHUNK, D), jnp.float32),
        pltpu.VMEM_SHARED((NPAD, D), jnp.float32),
        pltpu.SemaphoreType.DMA,
    ],
)
def _seg_sum(table, eidx, acc_out, idx_v, rows_v, acc_sh, sem):
    c = lax.axis_index("c")
    s = lax.axis_index("s")
    wid = c * 16 + s
    r0 = s * NSLICE
    for r in range(64):
        for h in range(D // 16):
            rows_v[0, r, pl.ds(h * 16, 16)] = jnp.zeros((16,), jnp.float32)
    for t, sz in _ZSLICES:
        pltpu.sync_copy(rows_v.at[0, pl.ds(0, sz)], acc_sh.at[pl.ds(r0 + t, sz)])
    plsc.subcore_barrier()

    def body(g, carry):
        pltpu.sync_copy(eidx.at[pl.ds((wid * C + g * G) * 2, 2 * G)], idx_v)
        cps = {}
        cps[0] = pltpu.async_copy(table.at[idx_v.at[0]], rows_v.at[0], sem)
        for j in range(G):
            if j + 1 < G:
                cps[j + 1] = pltpu.async_copy(
                    table.at[idx_v.at[2 * (j + 1)]], rows_v.at[(j + 1) % 2], sem)
            cps[j].wait()
            pltpu.sync_copy(rows_v.at[j % 2], acc_sh.at[idx_v.at[2 * j + 1]],
                            add=True)
        return carry

    lax.fori_loop(0, NG, body, 0)

    plsc.subcore_barrier()
    o0 = c * NPAD + s * NSLICE
    for t, sz in _SLICES:
        pltpu.sync_copy(acc_sh.at[pl.ds(r0 + t, sz)], rows_v.at[0, pl.ds(0, sz)])
        pltpu.sync_copy(rows_v.at[0, pl.ds(0, sz)], acc_out.at[pl.ds(o0 + t, sz)])


@functools.partial(
    pl.kernel,
    mesh=plsc.VectorSubcoreMesh(**_MESH),
    out_type=jax.ShapeDtypeStruct((2 * NPAD, D), jnp.float32),
    scratch_types=[
        pltpu.VMEM((2 * G, CHUNK), jnp.int32),
        pltpu.VMEM((CHUNK, D), jnp.float32),
        pltpu.VMEM_SHARED((NPAD, D), jnp.float32),
        pltpu.SemaphoreType.DMA,
    ],
)
def _deg_sum(eidx, ones, deg_out, idx_v, ones_v, deg_sh, sem):
    c = lax.axis_index("c")
    s = lax.axis_index("s")
    wid = c * 16 + s
    r0 = s * NSLICE
    for r in range(64):
        for h in range(D // 16):
            ones_v[r, pl.ds(h * 16, 16)] = jnp.zeros((16,), jnp.float32)
    for t, sz in _ZSLICES:
        pltpu.sync_copy(ones_v.at[pl.ds(0, sz)], deg_sh.at[pl.ds(r0 + t, sz)])
    pltpu.sync_copy(ones, ones_v)
    plsc.subcore_barrier()

    def body(g, carry):
        pltpu.sync_copy(eidx.at[pl.ds((wid * C + g * G) * 2, 2 * G)], idx_v)
        cps = [pltpu.async_copy(ones_v, deg_sh.at[idx_v.at[2 * j + 1]], sem,
                                add=True)
               for j in range(G)]
        for cp in cps:
            cp.wait()
        return carry

    lax.fori_loop(0, NG, body, 0)

    plsc.subcore_barrier()
    o0 = c * NPAD + s * NSLICE
    for t, sz in _SLICES:
        pltpu.sync_copy(deg_sh.at[pl.ds(r0 + t, sz)], ones_v.at[pl.ds(0, sz)])
        pltpu.sync_copy(ones_v.at[pl.ds(0, sz)], deg_out.at[pl.ds(o0 + t, sz)])


_MMB = 2000


def _mm2_body(x_ref, wl_ref, wr_ref, p_ref, r_ref):
    x = x_ref[...]
    p_ref[...] = jnp.dot(x, wl_ref[...], preferred_element_type=jnp.float32)
    r_ref[...] = jnp.dot(x, wr_ref[...], preferred_element_type=jnp.float32)


def _mm2(x, wl, wr):
    return pl.pallas_call(
        _mm2_body,
        grid=(N // _MMB,),
        in_specs=[
            pl.BlockSpec((_MMB, D), lambda i: (i, 0)),
            pl.BlockSpec((D, D), lambda i: (0, 0)),
            pl.BlockSpec((D, D), lambda i: (0, 0)),
        ],
        out_specs=[pl.BlockSpec((_MMB, D), lambda i: (i, 0))] * 2,
        out_shape=[jax.ShapeDtypeStruct((N, D), jnp.float32)] * 2,
    )(x, wl, wr)


def _fuse_body(acc_ref, deg_ref, r1_ref, b_ref, wl_ref, wr_ref, p2_ref, r2_ref):
    a = acc_ref[0] + acc_ref[1]
    dcol = deg_ref[0, :, :1] + deg_ref[1, :, :1]
    inv = 1.0 / jnp.maximum(dcol, 1.0)
    h = jnp.maximum(a * inv + b_ref[...] + r1_ref[...], 0.0)
    p2_ref[...] = jnp.dot(h, wl_ref[...], preferred_element_type=jnp.float32)
    r2_ref[...] = jnp.dot(h, wr_ref[...], preferred_element_type=jnp.float32)


def _fuse(acc, deg, r1, b1, wl, wr):
    return pl.pallas_call(
        _fuse_body,
        grid=(N // _MMB,),
        in_specs=[
            pl.BlockSpec((2, _MMB, D), lambda i: (0, i, 0)),
            pl.BlockSpec((2, _MMB, D), lambda i: (0, i, 0)),
            pl.BlockSpec((_MMB, D), lambda i: (i, 0)),
            pl.BlockSpec((1, D), lambda i: (0, 0)),
            pl.BlockSpec((D, D), lambda i: (0, 0)),
            pl.BlockSpec((D, D), lambda i: (0, 0)),
        ],
        out_specs=[pl.BlockSpec((_MMB, D), lambda i: (i, 0))] * 2,
        out_shape=[jax.ShapeDtypeStruct((N, D), jnp.float32)] * 2,
    )(acc, deg, r1, b1, wl, wr)


def _final_body(acc_ref, deg_ref, r2_ref, b_ref, out_ref):
    a = acc_ref[0] + acc_ref[1]
    dcol = deg_ref[0, :, :1] + deg_ref[1, :, :1]
    inv = 1.0 / jnp.maximum(dcol, 1.0)
    out_ref[...] = a * inv + b_ref[...] + r2_ref[...]


def _final(acc, deg, r2, b2):
    return pl.pallas_call(
        _final_body,
        grid=(N // _MMB,),
        in_specs=[
            pl.BlockSpec((2, _MMB, D), lambda i: (0, i, 0)),
            pl.BlockSpec((2, _MMB, D), lambda i: (0, i, 0)),
            pl.BlockSpec((_MMB, D), lambda i: (i, 0)),
            pl.BlockSpec((1, D), lambda i: (0, 0)),
        ],
        out_specs=pl.BlockSpec((_MMB, D), lambda i: (i, 0)),
        out_shape=jax.ShapeDtypeStruct((N, D), jnp.float32),
    )(acc, deg, r2, b2)


def kernel(x, edge_index, W1_l, b1_l, W1_r, W2_l, b2_l, W2_r):
    pad = EPAD - E
    ar = jnp.arange(pad, dtype=jnp.int32)
    src = jnp.concatenate([edge_index[0].astype(jnp.int32), ar % N])
    dst = jnp.concatenate(
        [edge_index[1].astype(jnp.int32), N + ar % (NPAD - N)])
    eidx = jnp.stack([src.reshape(NW * C, CHUNK), dst.reshape(NW * C, CHUNK)],
                     axis=1).reshape(2 * NW * C, CHUNK)
    ones = jnp.ones((CHUNK, D), jnp.float32)
    b1 = b1_l.reshape(1, D)
    b2 = b2_l.reshape(1, D)

    degr = _deg_sum(eidx, ones)
    degp = degr.reshape(2, NPAD, D)
    p1, r1 = _mm2(x, W1_l, W1_r)
    marker = (degr[0, 0] * 0.0).astype(jnp.int32)
    acc1 = _seg_sum(p1, eidx + marker).reshape(2, NPAD, D)
    p2, r2 = _fuse(acc1, degp, r1, b1, W2_l, W2_r)
    acc2 = _seg_sum(p2, eidx).reshape(2, NPAD, D)
    return _final(acc2, degp, r2, b2)
